# Initial kernel scaffold; baseline (speedup 1.0000x reference)
import jax
import jax.numpy as jnp
from jax.experimental import pallas as pl


def kernel(input_values, w0, w1, w2, w3, w4, w5, w6, codebooks):
    raise NotImplementedError("write your pallas kernel here")


# speedup vs baseline: 1.8139x; 1.8139x over previous
import jax
import jax.numpy as jnp
from jax.experimental import pallas as pl

_NQ = 8
_S = 120000
_T1 = 3840
_NT = 8
_L = 3968


def _elu(y):
    return jnp.where(y > 0, y, jnp.expm1(y))


def _dd(a, b):
    return jax.lax.dot_general(
        a, b, (((1,), (0,)), ((), ())), preferred_element_type=jnp.float32)


def _ddt(a, b):
    return jax.lax.dot_general(
        a, b, (((0,), (1,)), ((), ())), preferred_element_type=jnp.float32)


def _dot_exact(a, b):
    return jax.lax.dot_general(
        a, b, (((1,), (0,)), ((), ())),
        precision=jax.lax.Precision.HIGHEST,
        preferred_element_type=jnp.float32)


def _k0_body(xph_ref, w0f_ref, out_ref):
    t0 = pl.program_id(1) * _L
    rows = [xph_ref[0, c, pl.ds(t0, _L + 128)] for c in range(4)]
    for b in range(4):
        taps = jnp.concatenate(
            [rows[(b + 3 + j) % 4][None, (b + 3 + j) // 4:(b + 3 + j) // 4 + _L]
             for j in range(7)], axis=0)
        acc = _dd(w0f_ref[...], taps)
        m = t0 + jax.lax.broadcasted_iota(jnp.int32, (32, _L), 1)
        u = 4 * m + (b - 2)
        out_ref[0, b] = jnp.where((u >= 0) & (u < _S), acc, 0.0)


def _k1_body(hph_ref, w1f_ref, out_ref):
    t0 = pl.program_id(1) * _T1
    rows = [hph_ref[0, b, :, pl.ds(t0, _T1 + 128)] for b in range(4)]

    def tap(j):
        return rows[j % 4][:, j // 4:j // 4 + _T1]
    y1 = jnp.zeros((64, _T1), jnp.float32)
    for lo, hi in ((0, 2), (2, 6), (6, 8)):
        y1 = y1 + _dd(w1f_ref[:, 32 * lo:32 * hi],
                      jnp.concatenate([tap(j) for j in range(lo, hi)], axis=0))
    out_ref[0] = y1


def _k2_body(hph_ref, w2f_ref, out_ref):
    taps = [hph_ref[0, j % 5, :, pl.ds(j // 5, 6000)] for j in range(10)]
    main = _dd(w2f_ref[...],
               jnp.concatenate([t[:, :5888] for t in taps], axis=0))
    tail = _ddt(jnp.concatenate([t[:, 5888:] for t in taps], axis=0),
                w2f_ref[...])
    out_ref[0] = jnp.concatenate([main, tail.T], axis=1)


def _make_convT_body(kw, stride, mout):
    def body(hph_ref, wf_ref, out_ref):
        taps = jnp.concatenate(
            [hph_ref[0, j % stride, :, pl.ds(j // stride, mout)]
             for j in range(kw)], axis=0)
        out_ref[0] = _ddt(taps, wf_ref[...])
    return body


def _rvq_body(r0_ref, cb_ref, out_ref):
    r = r0_ref[...]
    for q in range(_NQ):
        cbq = cb_ref[q]
        rr = jnp.sum(r * r, axis=1, keepdims=True)
        cc = jnp.sum(cbq * cbq, axis=1)
        cross = jax.lax.dot_general(
            r.astype(jnp.bfloat16), cbq.astype(jnp.bfloat16),
            (((1,), (1,)), ((), ())),
            preferred_element_type=jnp.float32)
        d = rr - 2.0 * cross + cc[None, :]
        idx = jnp.argmin(d, axis=1).astype(jnp.int32)
        out_ref[q, :] = idx
        oh = (jax.lax.broadcasted_iota(jnp.int32, (256, 2048), 1)
              == idx[:, None]).astype(jnp.float32)
        r = r - _dot_exact(oh, cbq)


def _phase_split(h, stride, lo, hi):
    b, c, t = h.shape
    hp = jnp.pad(h, ((0, 0), (0, 0), (lo, hi)))
    m = (t + lo + hi) // stride
    return jnp.transpose(hp.reshape(b, c, m, stride), (0, 3, 1, 2))


def _wflat(w):
    return jnp.transpose(w, (0, 2, 1)).reshape(w.shape[0], -1).astype(jnp.bfloat16)


def kernel(input_values, w0, w1, w2, w3, w4, w5, w6, codebooks):
    bsz = input_values.shape[0]
    x = input_values[:, 0, :]

    w0f = w0[:, 0, :].astype(jnp.bfloat16)
    w1f = _wflat(w1)
    w2f = _wflat(w2)
    w3f = _wflat(w3)
    w4f = _wflat(w4)
    w5f = _wflat(w5)
    w6f = _wflat(w6)

    xp = jnp.pad(x, ((0, 0), (8, 7480)))
    xph = jnp.transpose(xp.reshape(bsz, 31872, 4), (0, 2, 1)
                        ).astype(jnp.bfloat16)
    y0 = pl.pallas_call(
        _k0_body,
        grid=(bsz, _NT),
        in_specs=[pl.BlockSpec((1, 4, 31872), lambda b, t: (b, 0, 0)),
                  pl.BlockSpec((32, 7), lambda b, t: (0, 0))],
        out_specs=pl.BlockSpec((1, 4, 32, _L), lambda b, t: (b, 0, 0, t)),
        out_shape=jax.ShapeDtypeStruct((bsz, 4, 32, _NT * _L), jnp.float32),
    )(xph, w0f)
    y0 = _elu(y0).astype(jnp.bfloat16)

    y1 = pl.pallas_call(
        _k1_body,
        grid=(bsz, _NT),
        in_specs=[pl.BlockSpec((1, 4, 32, 31744), lambda b, t: (b, 0, 0, 0)),
                  pl.BlockSpec((64, 256), lambda b, t: (0, 0))],
        out_specs=pl.BlockSpec((1, 64, _T1), lambda b, t: (b, 0, t)),
        out_shape=jax.ShapeDtypeStruct((bsz, 64, _NT * _T1), jnp.float32),
    )(y0, w1f)
    y1 = _elu(y1[:, :, :30000]).astype(jnp.bfloat16)

    y1ph = _phase_split(y1, 5, 2, 3)
    y2 = pl.pallas_call(
        _k2_body,
        grid=(bsz,),
        in_specs=[pl.BlockSpec((1, 5, 64, 6001), lambda b: (b, 0, 0, 0)),
                  pl.BlockSpec((128, 640), lambda b: (0, 0))],
        out_specs=pl.BlockSpec((1, 128, 6000), lambda b: (b, 0, 0)),
        out_shape=jax.ShapeDtypeStruct((bsz, 128, 6000), jnp.float32),
    )(y1ph, w2f)
    y2 = _elu(y2).astype(jnp.bfloat16)

    y2ph = _phase_split(y2, 6, 3, 3)
    y3t = pl.pallas_call(
        _make_convT_body(12, 6, 1000),
        grid=(bsz,),
        in_specs=[pl.BlockSpec((1, 6, 128, 1001), lambda b: (b, 0, 0, 0)),
                  pl.BlockSpec((256, 1536), lambda b: (0, 0))],
        out_specs=pl.BlockSpec((1, 1000, 256), lambda b: (b, 0, 0)),
        out_shape=jax.ShapeDtypeStruct((bsz, 1000, 256), jnp.float32),
    )(y2ph, w3f)
    h3 = jnp.transpose(_elu(y3t).astype(jnp.bfloat16), (0, 2, 1))

    y3ph = _phase_split(h3, 8, 4, 4)
    y4t = pl.pallas_call(
        _make_convT_body(16, 8, 125),
        grid=(bsz,),
        in_specs=[pl.BlockSpec((1, 8, 256, 126), lambda b: (b, 0, 0, 0)),
                  pl.BlockSpec((512, 4096), lambda b: (0, 0))],
        out_specs=pl.BlockSpec((1, 125, 512), lambda b: (b, 0, 0)),
        out_shape=jax.ShapeDtypeStruct((bsz, 125, 512), jnp.float32),
    )(y3ph, w4f)

    h4 = jnp.transpose(_elu(y4t).astype(jnp.bfloat16), (0, 2, 1))
    y4p = jnp.pad(h4, ((0, 0), (0, 0), (1, 1)))[:, None]
    y5t = pl.pallas_call(
        _make_convT_body(3, 1, 125),
        grid=(bsz,),
        in_specs=[pl.BlockSpec((1, 1, 512, 127), lambda b: (b, 0, 0, 0)),
                  pl.BlockSpec((512, 1536), lambda b: (0, 0))],
        out_specs=pl.BlockSpec((1, 125, 512), lambda b: (b, 0, 0)),
        out_shape=jax.ShapeDtypeStruct((bsz, 125, 512), jnp.float32),
    )(y4p, w5f)

    y5p = jnp.pad(_elu(y5t).astype(jnp.bfloat16),
                  ((0, 0), (1, 2), (0, 0)))
    y5ph = jnp.transpose(y5p.reshape(bsz, 64, 2, 512), (0, 2, 3, 1))
    emb_t = pl.pallas_call(
        _make_convT_body(4, 2, 63),
        grid=(bsz,),
        in_specs=[pl.BlockSpec((1, 2, 512, 64), lambda b: (b, 0, 0, 0)),
                  pl.BlockSpec((512, 2048), lambda b: (0, 0))],
        out_specs=pl.BlockSpec((1, 63, 512), lambda b: (b, 0, 0)),
        out_shape=jax.ShapeDtypeStruct((bsz, 63, 512), jnp.float32),
    )(y5ph, w6f)

    r0 = emb_t.reshape(bsz * 63, 512)
    r0 = jnp.pad(r0, ((0, 256 - bsz * 63), (0, 0)))
    codes = pl.pallas_call(
        _rvq_body,
        grid=(1,),
        in_specs=[pl.BlockSpec((256, 512), lambda i: (0, 0)),
                  pl.BlockSpec((_NQ, 2048, 512), lambda i: (0, 0, 0))],
        out_specs=pl.BlockSpec((_NQ, 256), lambda i: (0, 0)),
        out_shape=jax.ShapeDtypeStruct((_NQ, 256), jnp.int32),
    )(r0, codebooks)
    return jnp.transpose(
        codes[:, :bsz * 63].reshape(_NQ, bsz, 63), (1, 0, 2)).astype(jnp.int32)
